# baseline (speedup 1.0000x reference)
import functools

import jax
import jax.numpy as jnp
from jax.experimental import pallas as pl
from jax.experimental.pallas import tpu as pltpu
from jax.experimental.pallas import tpu_sc as plsc

_MESH = plsc.ScalarSubcoreMesh(axis_name="c", num_cores=1)


@functools.partial(
    pl.kernel,
    mesh=_MESH,
    out_type=jax.ShapeDtypeStruct((1,), jnp.float32),
    scratch_types=[
        pltpu.SMEM((1,), jnp.float32),
        pltpu.SMEM((16,), jnp.float32),
        pltpu.SMEM((4, 15), jnp.float32),
        pltpu.SMEM((1,), jnp.float32),
    ],
    compiler_params=pltpu.CompilerParams(needs_layout_passes=False),
)
def _akima_scs(b_hbm, xs_hbm, c_hbm, out_hbm, b_s, xs_s, c_s, o_s):
    pltpu.sync_copy(b_hbm, b_s)
    pltpu.sync_copy(xs_hbm, xs_s)
    pltpu.sync_copy(c_hbm, c_s)
    x = b_s[0]
    cnt = jnp.int32(0)
    for j in range(16):
        cnt = cnt + jnp.where(xs_s[j] <= x, jnp.int32(1), jnp.int32(0))
    i = jnp.clip(cnt - 1, 0, 14)
    bx = x - xs_s[i]
    v = c_s[3, i] + bx * (c_s[2, i] + bx * (c_s[1, i] + bx * c_s[0, i]))
    o_s[0] = jnp.where(cnt < 16, v, jnp.float32(0.0))
    pltpu.sync_copy(o_s, out_hbm)


def kernel(b, xs, c):
    return _akima_scs(b, xs, c)[0]


# speedup vs baseline: 1.0597x; 1.0597x over previous
import functools

import jax
import jax.numpy as jnp
from jax.experimental import pallas as pl
from jax.experimental.pallas import tpu as pltpu
from jax.experimental.pallas import tpu_sc as plsc

_MESH = plsc.ScalarSubcoreMesh(axis_name="c", num_cores=1)


@functools.partial(
    pl.kernel,
    mesh=_MESH,
    out_type=jax.ShapeDtypeStruct((1,), jnp.float32),
    scratch_types=[
        pltpu.SMEM((1,), jnp.float32),
        pltpu.SMEM((16,), jnp.float32),
        pltpu.SMEM((4, 15), jnp.float32),
        pltpu.SMEM((1,), jnp.float32),
        pltpu.SemaphoreType.DMA,
        pltpu.SemaphoreType.DMA,
        pltpu.SemaphoreType.DMA,
    ],
    compiler_params=pltpu.CompilerParams(needs_layout_passes=False),
)
def _akima_scs(b_hbm, xs_hbm, c_hbm, out_hbm, b_s, xs_s, c_s, o_s, s1, s2, s3):
    cp1 = pltpu.async_copy(b_hbm, b_s, s1)
    cp2 = pltpu.async_copy(xs_hbm, xs_s, s2)
    cp3 = pltpu.async_copy(c_hbm, c_s, s3)
    cp1.wait()
    cp2.wait()
    cp3.wait()
    x = b_s[0]
    cnt = jnp.int32(0)
    for j in range(16):
        cnt = cnt + jnp.where(xs_s[j] <= x, jnp.int32(1), jnp.int32(0))
    i = jnp.clip(cnt - 1, 0, 14)
    bx = x - xs_s[i]
    v = c_s[3, i] + bx * (c_s[2, i] + bx * (c_s[1, i] + bx * c_s[0, i]))
    o_s[0] = jnp.where(cnt < 16, v, jnp.float32(0.0))
    pltpu.sync_copy(o_s, out_hbm)


def kernel(b, xs, c):
    return _akima_scs(b, xs, c)[0]
